# baseline (speedup 1.0000x reference)
import functools

import jax
import jax.numpy as jnp
from jax import lax
from jax.experimental import pallas as pl
from jax.experimental.pallas import tpu as pltpu
from jax.experimental.pallas import tpu_sc as plsc

B = 1024
D_IN = 2048
D_OUT = 256

NUM_CORES = 1
NUM_SUBCORES = 16
LANES = 16
NW = NUM_CORES * NUM_SUBCORES
ROWS_PER = B // NW
CHUNKS = B // LANES


BLK = 256
NRB = B // BLK


def _fc_body(x_ref, w_ref, b_ref, h_ref):
    h_ref[...] = (
        jnp.dot(x_ref[...], w_ref[...], preferred_element_type=jnp.float32)
        + b_ref[...]
    )


def _d2_body(hb_ref, h_ref, out_ref):
    hb = hb_ref[...]
    hall = h_ref[...]
    hh = hall * hall
    ones_row = jnp.ones((1, D_OUT), jnp.float32)
    sq_row = lax.dot_general(ones_row, hh, (((1,), (1,)), ((), ())),
                             preferred_element_type=jnp.float32)
    sq_blk = jnp.sum(hb * hb, axis=1, keepdims=True)
    for t in range(NRB):
        hc = hall[t * BLK:(t + 1) * BLK, :]
        g = lax.dot_general(hb, hc, (((1,), (1,)), ((), ())),
                            preferred_element_type=jnp.float32)
        d2p = sq_blk + sq_row[:, t * BLK:(t + 1) * BLK] - 2.0 * g
        d2p = jnp.maximum(d2p, 0.0)
        for u in range(BLK // 128):
            out_ref[:, (t * BLK) // 128 + u, :] = d2p[:, u * 128:(u + 1) * 128]


RBLK = 4
NBLK = ROWS_PER // RBLK


def _mine_body(d2_hbm, tgt_hbm, mp_hbm, mn_hbm, d2_v, tgt_v, mp_v, mn_v):
    wid = lax.axis_index("s") * NUM_CORES + lax.axis_index("c")
    base = wid * ROWS_PER
    pltpu.sync_copy(d2_hbm.at[pl.ds(base * B, ROWS_PER * B)], d2_v)
    pltpu.sync_copy(tgt_hbm, tgt_v.at[pl.ds(0, B)])

    def blk_body(blk, _):
        r0 = blk * RBLK
        ts = [
            jnp.full((LANES,), tgt_v[pl.ds(base + r0 + i, LANES)][0],
                     jnp.int32)
            for i in range(RBLK)
        ]
        mp = [jnp.full((LANES,), -jnp.inf, jnp.float32)] * RBLK
        mn = [jnp.full((LANES,), jnp.inf, jnp.float32)] * RBLK
        for j in range(CHUNKS):
            tv = tgt_v[pl.ds(j * LANES, LANES)]
            for i in range(RBLK):
                dv = d2_v[pl.ds((r0 + i) * B + j * LANES, LANES)]
                same = tv == ts[i]
                mp[i] = jnp.maximum(mp[i], jnp.where(same, dv, -jnp.inf))
                mn[i] = jnp.minimum(mn[i], jnp.where(same, jnp.inf, dv))
        for i in range(RBLK):
            mp_v[pl.ds((r0 + i) * LANES, LANES)] = mp[i]
            mn_v[pl.ds((r0 + i) * LANES, LANES)] = mn[i]
        return 0

    lax.fori_loop(0, NBLK, blk_body, 0)
    pltpu.sync_copy(mp_v, mp_hbm.at[pl.ds(base * LANES, ROWS_PER * LANES)])
    pltpu.sync_copy(mn_v, mn_hbm.at[pl.ds(base * LANES, ROWS_PER * LANES)])


@functools.lru_cache(maxsize=1)
def _mine_kernel():
    return pl.kernel(
        _mine_body,
        out_type=(
            jax.ShapeDtypeStruct((B * LANES,), jnp.float32),
            jax.ShapeDtypeStruct((B * LANES,), jnp.float32),
        ),
        mesh=plsc.VectorSubcoreMesh(core_axis_name="c", subcore_axis_name="s",
                                    num_cores=NUM_CORES,
                                    num_subcores=NUM_SUBCORES),
        scratch_types=[
            pltpu.VMEM((ROWS_PER * B,), jnp.float32),
            pltpu.VMEM((B + LANES,), jnp.int32),
            pltpu.VMEM((ROWS_PER * LANES,), jnp.float32),
            pltpu.VMEM((ROWS_PER * LANES,), jnp.float32),
        ],
    )


def _loss_body(mp_ref, mn_ref, out_ref):
    md2 = jnp.max(mp_ref[...], axis=1, keepdims=True)
    nd2 = jnp.min(mn_ref[...], axis=1, keepdims=True)
    dp = jnp.where(md2 > 1e-12, jnp.sqrt(jnp.where(md2 > 1e-12, md2, 1.0)), 0.0)
    dn = jnp.where(nd2 > 1e-12, jnp.sqrt(jnp.where(nd2 > 1e-12, nd2, 1.0)), 0.0)
    out_ref[0, 0] = jnp.sum(jnp.log1p(jnp.exp(dp - dn)))


def kernel(inputs, targets, W, b):
    h = pl.pallas_call(
        _fc_body,
        grid=(NRB,),
        in_specs=[
            pl.BlockSpec((BLK, D_IN), lambda i: (i, 0)),
            pl.BlockSpec((D_IN, D_OUT), lambda i: (0, 0)),
            pl.BlockSpec((1, D_OUT), lambda i: (0, 0)),
        ],
        out_specs=pl.BlockSpec((BLK, D_OUT), lambda i: (i, 0)),
        out_shape=jax.ShapeDtypeStruct((B, D_OUT), jnp.float32),
    )(inputs, W, b.reshape(1, D_OUT))

    d2 = pl.pallas_call(
        _d2_body,
        grid=(NRB,),
        in_specs=[
            pl.BlockSpec((BLK, D_OUT), lambda i: (i, 0)),
            pl.BlockSpec((B, D_OUT), lambda i: (0, 0)),
        ],
        out_specs=pl.BlockSpec((BLK, 8, 128), lambda i: (i, 0, 0)),
        out_shape=jax.ShapeDtypeStruct((B, 8, 128), jnp.float32),
    )(h, h)

    mp, mn = _mine_kernel()(d2.reshape(B * B), targets)

    loss = pl.pallas_call(
        _loss_body,
        out_shape=jax.ShapeDtypeStruct((1, 1), jnp.float32),
        out_specs=pl.BlockSpec(memory_space=pltpu.SMEM),
    )(mp.reshape(B, LANES), mn.reshape(B, LANES))
    return loss.reshape(())


# speedup vs baseline: 1.1283x; 1.1283x over previous
import functools

import jax
import jax.numpy as jnp
from jax import lax
from jax.experimental import pallas as pl
from jax.experimental.pallas import tpu as pltpu
from jax.experimental.pallas import tpu_sc as plsc

B = 1024
D_IN = 2048
D_OUT = 256

NUM_CORES = 2
NUM_SUBCORES = 16
LANES = 16
NW = NUM_CORES * NUM_SUBCORES
ROWS_PER = B // NW
CHUNKS = B // LANES


BLK = 256
NRB = B // BLK


def _fc_body(x_ref, w_ref, b_ref, h_ref):
    h_ref[...] = (
        jnp.dot(x_ref[...], w_ref[...], preferred_element_type=jnp.float32)
        + b_ref[...]
    )


def _d2_body(hb_ref, h_ref, out_ref):
    hb = hb_ref[...]
    hall = h_ref[...]
    hh = hall * hall
    ones_row = jnp.ones((1, D_OUT), jnp.float32)
    sq_row = lax.dot_general(ones_row, hh, (((1,), (1,)), ((), ())),
                             preferred_element_type=jnp.float32)
    sq_blk = jnp.sum(hb * hb, axis=1, keepdims=True)
    for t in range(NRB):
        hc = hall[t * BLK:(t + 1) * BLK, :]
        g = lax.dot_general(hb, hc, (((1,), (1,)), ((), ())),
                            preferred_element_type=jnp.float32)
        d2p = sq_blk + sq_row[:, t * BLK:(t + 1) * BLK] - 2.0 * g
        d2p = jnp.maximum(d2p, 0.0)
        for u in range(BLK // 128):
            out_ref[:, (t * BLK) // 128 + u, :] = d2p[:, u * 128:(u + 1) * 128]


RBLK = 4
NBLK = ROWS_PER // RBLK


def _mine_body(d2_hbm, tgt_hbm, mp_hbm, mn_hbm, d2_v, tgt_v, mp_v, mn_v):
    wid = lax.axis_index("s") * NUM_CORES + lax.axis_index("c")
    base = wid * ROWS_PER
    pltpu.sync_copy(d2_hbm.at[pl.ds(base * B, ROWS_PER * B)], d2_v)
    pltpu.sync_copy(tgt_hbm, tgt_v.at[pl.ds(0, B)])
    def blk_body(blk, _):
        r0 = blk * RBLK
        ts = [
            jnp.full((LANES,), tgt_v[pl.ds(base + r0 + i, LANES)][0],
                     jnp.int32)
            for i in range(RBLK)
        ]
        mp = [jnp.full((LANES,), -jnp.inf, jnp.float32)] * RBLK
        mn = [jnp.full((LANES,), jnp.inf, jnp.float32)] * RBLK
        for j in range(CHUNKS):
            tv = tgt_v[pl.ds(j * LANES, LANES)]
            for i in range(RBLK):
                dv = d2_v[pl.ds((r0 + i) * B + j * LANES, LANES)]
                same = tv == ts[i]
                mp[i] = jnp.maximum(mp[i], jnp.where(same, dv, -jnp.inf))
                mn[i] = jnp.minimum(mn[i], jnp.where(same, jnp.inf, dv))
        for i in range(RBLK):
            mp_v[pl.ds((r0 + i) * LANES, LANES)] = mp[i]
            mn_v[pl.ds((r0 + i) * LANES, LANES)] = mn[i]
        return 0

    lax.fori_loop(0, NBLK, blk_body, 0)
    pltpu.sync_copy(mp_v, mp_hbm.at[pl.ds(base * LANES, ROWS_PER * LANES)])
    pltpu.sync_copy(mn_v, mn_hbm.at[pl.ds(base * LANES, ROWS_PER * LANES)])


@functools.lru_cache(maxsize=1)
def _mine_kernel():
    return pl.kernel(
        _mine_body,
        out_type=(
            jax.ShapeDtypeStruct((B * LANES,), jnp.float32),
            jax.ShapeDtypeStruct((B * LANES,), jnp.float32),
        ),
        mesh=plsc.VectorSubcoreMesh(core_axis_name="c", subcore_axis_name="s",
                                    num_cores=NUM_CORES,
                                    num_subcores=NUM_SUBCORES),
        scratch_types=[
            pltpu.VMEM((ROWS_PER * B,), jnp.float32),
            pltpu.VMEM((B + LANES,), jnp.int32),
            pltpu.VMEM((ROWS_PER * LANES,), jnp.float32),
            pltpu.VMEM((ROWS_PER * LANES,), jnp.float32),
        ],
    )


def _loss_body(mp_ref, mn_ref, out_ref):
    mp = mp_ref[...]
    mn = mn_ref[...]
    acc = jnp.zeros((128, 1), jnp.float32)
    for g in range(8):
        md2 = jnp.max(mp[:, g * 16:(g + 1) * 16], axis=1, keepdims=True)
        nd2 = jnp.min(mn[:, g * 16:(g + 1) * 16], axis=1, keepdims=True)
        dp = jnp.where(md2 > 1e-12,
                       jnp.sqrt(jnp.where(md2 > 1e-12, md2, 1.0)), 0.0)
        dn = jnp.where(nd2 > 1e-12,
                       jnp.sqrt(jnp.where(nd2 > 1e-12, nd2, 1.0)), 0.0)
        acc = acc + jnp.log1p(jnp.exp(dp - dn))
    out_ref[0, 0] = jnp.sum(acc)


def kernel(inputs, targets, W, b):
    h = pl.pallas_call(
        _fc_body,
        grid=(NRB,),
        in_specs=[
            pl.BlockSpec((BLK, D_IN), lambda i: (i, 0)),
            pl.BlockSpec((D_IN, D_OUT), lambda i: (0, 0)),
            pl.BlockSpec((1, D_OUT), lambda i: (0, 0)),
        ],
        out_specs=pl.BlockSpec((BLK, D_OUT), lambda i: (i, 0)),
        out_shape=jax.ShapeDtypeStruct((B, D_OUT), jnp.float32),
    )(inputs, W, b.reshape(1, D_OUT))

    d2 = pl.pallas_call(
        _d2_body,
        grid=(NRB,),
        in_specs=[
            pl.BlockSpec((BLK, D_OUT), lambda i: (i, 0)),
            pl.BlockSpec((B, D_OUT), lambda i: (0, 0)),
        ],
        out_specs=pl.BlockSpec((BLK, 8, 128), lambda i: (i, 0, 0)),
        out_shape=jax.ShapeDtypeStruct((B, 8, 128), jnp.float32),
    )(h, h)

    mp, mn = _mine_kernel()(d2.reshape(B * B), targets)

    loss = pl.pallas_call(
        _loss_body,
        out_shape=jax.ShapeDtypeStruct((1, 1), jnp.float32),
        out_specs=pl.BlockSpec(memory_space=pltpu.SMEM),
    )(mp.reshape(128, 128), mn.reshape(128, 128))
    return loss.reshape(())


# speedup vs baseline: 1.1877x; 1.0527x over previous
import functools

import jax
import jax.numpy as jnp
from jax import lax
from jax.experimental import pallas as pl
from jax.experimental.pallas import tpu as pltpu
from jax.experimental.pallas import tpu_sc as plsc

B = 1024
D_IN = 2048
D_OUT = 256

NUM_CORES = 2
NUM_SUBCORES = 16
LANES = 16
NW = NUM_CORES * NUM_SUBCORES
ROWS_PER = B // NW
CHUNKS = B // LANES


BLK = 256
NRB = B // BLK


def _mega_body(x_hbm, w_hbm, b_hbm, out_hbm,
               x_v, w_v, b_v, h_v, buf0, buf1,
               sems_x, sem_w, sem_b, sems_o):
    cw = pltpu.make_async_copy(w_hbm, w_v, sem_w)
    cw.start()
    cb = pltpu.make_async_copy(b_hbm, b_v, sem_b)
    cb.start()
    cxs = []
    for c in range(NRB):
        cx = pltpu.make_async_copy(x_hbm.at[pl.ds(c * BLK, BLK)],
                                   x_v.at[pl.ds(c * BLK, BLK)],
                                   sems_x.at[c])
        cx.start()
        cxs.append(cx)
    cw.wait()
    cb.wait()
    bias = b_v[...]
    for c in range(NRB):
        cxs[c].wait()
        h_v[pl.ds(c * BLK, BLK), :] = (
            jnp.dot(x_v[pl.ds(c * BLK, BLK), :], w_v[...],
                    preferred_element_type=jnp.float32) + bias
        )
    h = h_v[...]
    hh = h * h
    ones_row = jnp.ones((1, D_OUT), jnp.float32)
    sq_row = lax.dot_general(ones_row, hh, (((1,), (1,)), ((), ())),
                             preferred_element_type=jnp.float32)
    bufs = (buf0, buf1)
    outcps = []
    for rb in range(NRB):
        buf = bufs[rb % 2]
        if rb >= 2:
            outcps[rb - 2].wait()
        hb = h[rb * BLK:(rb + 1) * BLK, :]
        sq_blk = jnp.sum(hb * hb, axis=1, keepdims=True)
        for t in range(NRB):
            hc = h[t * BLK:(t + 1) * BLK, :]
            g = lax.dot_general(hb, hc, (((1,), (1,)), ((), ())),
                                preferred_element_type=jnp.float32)
            d2p = sq_blk + sq_row[:, t * BLK:(t + 1) * BLK] - 2.0 * g
            d2p = jnp.maximum(d2p, 0.0)
            for u in range(BLK // 128):
                buf[:, (t * BLK) // 128 + u, :] = d2p[:, u * 128:(u + 1) * 128]
        cp = pltpu.make_async_copy(buf, out_hbm.at[pl.ds(rb * BLK, BLK)],
                                   sems_o.at[rb % 2])
        cp.start()
        outcps.append(cp)
    outcps[NRB - 2].wait()
    outcps[NRB - 1].wait()


RBLK = 4
NBLK = ROWS_PER // RBLK


def _mine_body(d2_hbm, tgt_hbm, mp_hbm, mn_hbm, d2_v, tgt_v, mp_v, mn_v):
    wid = lax.axis_index("s") * NUM_CORES + lax.axis_index("c")
    base = wid * ROWS_PER
    pltpu.sync_copy(d2_hbm.at[pl.ds(base * B, ROWS_PER * B)], d2_v)
    pltpu.sync_copy(tgt_hbm, tgt_v.at[pl.ds(0, B)])
    def blk_body(blk, _):
        r0 = blk * RBLK
        ts = [
            jnp.full((LANES,), tgt_v[pl.ds(base + r0 + i, LANES)][0],
                     jnp.int32)
            for i in range(RBLK)
        ]
        mp = [jnp.full((LANES,), -jnp.inf, jnp.float32)] * RBLK
        mn = [jnp.full((LANES,), jnp.inf, jnp.float32)] * RBLK
        for j in range(CHUNKS):
            tv = tgt_v[pl.ds(j * LANES, LANES)]
            for i in range(RBLK):
                dv = d2_v[pl.ds((r0 + i) * B + j * LANES, LANES)]
                same = tv == ts[i]
                mp[i] = jnp.maximum(mp[i], jnp.where(same, dv, -jnp.inf))
                mn[i] = jnp.minimum(mn[i], jnp.where(same, jnp.inf, dv))
        for i in range(RBLK):
            mp_v[pl.ds((r0 + i) * LANES, LANES)] = mp[i]
            mn_v[pl.ds((r0 + i) * LANES, LANES)] = mn[i]
        return 0

    lax.fori_loop(0, NBLK, blk_body, 0)
    pltpu.sync_copy(mp_v, mp_hbm.at[pl.ds(base * LANES, ROWS_PER * LANES)])
    pltpu.sync_copy(mn_v, mn_hbm.at[pl.ds(base * LANES, ROWS_PER * LANES)])


@functools.lru_cache(maxsize=1)
def _mine_kernel():
    return pl.kernel(
        _mine_body,
        out_type=(
            jax.ShapeDtypeStruct((B * LANES,), jnp.float32),
            jax.ShapeDtypeStruct((B * LANES,), jnp.float32),
        ),
        mesh=plsc.VectorSubcoreMesh(core_axis_name="c", subcore_axis_name="s",
                                    num_cores=NUM_CORES,
                                    num_subcores=NUM_SUBCORES),
        scratch_types=[
            pltpu.VMEM((ROWS_PER * B,), jnp.float32),
            pltpu.VMEM((B + LANES,), jnp.int32),
            pltpu.VMEM((ROWS_PER * LANES,), jnp.float32),
            pltpu.VMEM((ROWS_PER * LANES,), jnp.float32),
        ],
    )


def _loss_body(mp_ref, mn_ref, out_ref):
    mp = mp_ref[...]
    mn = mn_ref[...]
    acc = jnp.zeros((128, 1), jnp.float32)
    for g in range(8):
        md2 = jnp.max(mp[:, g * 16:(g + 1) * 16], axis=1, keepdims=True)
        nd2 = jnp.min(mn[:, g * 16:(g + 1) * 16], axis=1, keepdims=True)
        dp = jnp.where(md2 > 1e-12,
                       jnp.sqrt(jnp.where(md2 > 1e-12, md2, 1.0)), 0.0)
        dn = jnp.where(nd2 > 1e-12,
                       jnp.sqrt(jnp.where(nd2 > 1e-12, nd2, 1.0)), 0.0)
        acc = acc + jnp.log1p(jnp.exp(dp - dn))
    out_ref[0, 0] = jnp.sum(acc)


def kernel(inputs, targets, W, b):
    d2 = pl.pallas_call(
        _mega_body,
        in_specs=[
            pl.BlockSpec(memory_space=pltpu.MemorySpace.HBM),
            pl.BlockSpec(memory_space=pltpu.MemorySpace.HBM),
            pl.BlockSpec(memory_space=pltpu.MemorySpace.HBM),
        ],
        out_specs=pl.BlockSpec(memory_space=pltpu.MemorySpace.HBM),
        out_shape=jax.ShapeDtypeStruct((B, 8, 128), jnp.float32),
        scratch_shapes=[
            pltpu.VMEM((B, D_IN), jnp.float32),
            pltpu.VMEM((D_IN, D_OUT), jnp.float32),
            pltpu.VMEM((1, D_OUT), jnp.float32),
            pltpu.VMEM((B, D_OUT), jnp.float32),
            pltpu.VMEM((BLK, 8, 128), jnp.float32),
            pltpu.VMEM((BLK, 8, 128), jnp.float32),
            pltpu.SemaphoreType.DMA((NRB,)),
            pltpu.SemaphoreType.DMA,
            pltpu.SemaphoreType.DMA,
            pltpu.SemaphoreType.DMA((2,)),
        ],
    )(inputs, W, b.reshape(1, D_OUT))

    mp, mn = _mine_kernel()(d2.reshape(B * B), targets)

    loss = pl.pallas_call(
        _loss_body,
        out_shape=jax.ShapeDtypeStruct((1, 1), jnp.float32),
        out_specs=pl.BlockSpec(memory_space=pltpu.SMEM),
    )(mp.reshape(128, 128), mn.reshape(128, 128))
    return loss.reshape(())


# speedup vs baseline: 1.2397x; 1.0438x over previous
import functools

import jax
import jax.numpy as jnp
from jax import lax
from jax.experimental import pallas as pl
from jax.experimental.pallas import tpu as pltpu
from jax.experimental.pallas import tpu_sc as plsc

B = 1024
D_IN = 2048
D_OUT = 256

NUM_CORES = 2
NUM_SUBCORES = 16
LANES = 16
NW = NUM_CORES * NUM_SUBCORES
ROWS_PER = B // NW
CHUNKS = B // LANES


BLK = 256
NRB = B // BLK


def _mega_body(x_hbm, w_hbm, b_hbm, out_hbm,
               x_v, w_v, b_v, h_v, buf0, buf1,
               sems_x, sem_w, sem_b, sems_o):
    cw = pltpu.make_async_copy(w_hbm, w_v, sem_w)
    cw.start()
    cb = pltpu.make_async_copy(b_hbm, b_v, sem_b)
    cb.start()
    cxs = []
    for c in range(NRB):
        cx = pltpu.make_async_copy(x_hbm.at[pl.ds(c * BLK, BLK)],
                                   x_v.at[pl.ds(c * BLK, BLK)],
                                   sems_x.at[c])
        cx.start()
        cxs.append(cx)
    cw.wait()
    cb.wait()
    bias = b_v[...]
    for c in range(NRB):
        cxs[c].wait()
        h_v[pl.ds(c * BLK, BLK), :] = (
            jnp.dot(x_v[pl.ds(c * BLK, BLK), :], w_v[...],
                    preferred_element_type=jnp.float32) + bias
        )
    h = h_v[...]
    hm = h * -2.0
    hh = h * h
    ones_row = jnp.ones((1, D_OUT), jnp.float32)
    sq_row = lax.dot_general(ones_row, hh, (((1,), (1,)), ((), ())),
                             preferred_element_type=jnp.float32)
    bufs = (buf0, buf1)
    outcps = []
    for rb in range(NRB):
        buf = bufs[rb % 2]
        if rb >= 2:
            outcps[rb - 2].wait()
        hb = hm[rb * BLK:(rb + 1) * BLK, :]
        sq_blk = jnp.sum(h[rb * BLK:(rb + 1) * BLK, :] ** 2, axis=1,
                         keepdims=True)
        for t in range(NRB):
            hc = h[t * BLK:(t + 1) * BLK, :]
            g = lax.dot_general(hb, hc, (((1,), (1,)), ((), ())),
                                preferred_element_type=jnp.float32)
            d2p = jnp.maximum(sq_blk + (sq_row[:, t * BLK:(t + 1) * BLK] + g),
                              0.0)
            for u in range(BLK // 128):
                buf[:, (t * BLK) // 128 + u, :, :] = (
                    d2p[:, u * 128:(u + 1) * 128].reshape(BLK // 8, 8, 128))
        cp = pltpu.make_async_copy(buf, out_hbm.at[pl.ds(rb * (BLK // 8),
                                                         BLK // 8)],
                                   sems_o.at[rb % 2])
        cp.start()
        outcps.append(cp)
    outcps[NRB - 2].wait()
    outcps[NRB - 1].wait()


RBLK = 4
NBLK = ROWS_PER // 8


def _mine_body(d2_hbm, tgt_hbm, mp_hbm, mn_hbm, d2_v, tgt_v, mp_v, mn_v):
    wid = lax.axis_index("s") * NUM_CORES + lax.axis_index("c")
    base = wid * ROWS_PER
    pltpu.sync_copy(d2_hbm.at[pl.ds(base * B, ROWS_PER * B)], d2_v)
    pltpu.sync_copy(tgt_hbm, tgt_v.at[pl.ds(0, B)])

    def blk_body(blk, _):
        dbase = blk * (8 * B)
        for half in range(2):
            r0 = blk * 8 + half * RBLK
            ts = [
                jnp.full((LANES,), tgt_v[pl.ds(base + r0 + i, LANES)][0],
                         jnp.int32)
                for i in range(RBLK)
            ]
            mp = [jnp.full((LANES,), -jnp.inf, jnp.float32)] * RBLK
            mn = [jnp.full((LANES,), jnp.inf, jnp.float32)] * RBLK
            for j in range(CHUNKS):
                tv = tgt_v[pl.ds(j * LANES, LANES)]
                joff = (j // 8) * B + (j % 8) * LANES
                for i in range(RBLK):
                    dv = d2_v[pl.ds(dbase + (half * RBLK + i) * 128 + joff,
                                    LANES)]
                    same = tv == ts[i]
                    mp[i] = jnp.maximum(mp[i], jnp.where(same, dv, -jnp.inf))
                    mn[i] = jnp.minimum(mn[i], jnp.where(same, jnp.inf, dv))
            for i in range(RBLK):
                mp_v[pl.ds((r0 + i) * LANES, LANES)] = mp[i]
                mn_v[pl.ds((r0 + i) * LANES, LANES)] = mn[i]
        return 0

    lax.fori_loop(0, NBLK, blk_body, 0)
    pltpu.sync_copy(mp_v, mp_hbm.at[pl.ds(base * LANES, ROWS_PER * LANES)])
    pltpu.sync_copy(mn_v, mn_hbm.at[pl.ds(base * LANES, ROWS_PER * LANES)])


@functools.lru_cache(maxsize=1)
def _mine_kernel():
    return pl.kernel(
        _mine_body,
        out_type=(
            jax.ShapeDtypeStruct((B * LANES,), jnp.float32),
            jax.ShapeDtypeStruct((B * LANES,), jnp.float32),
        ),
        mesh=plsc.VectorSubcoreMesh(core_axis_name="c", subcore_axis_name="s",
                                    num_cores=NUM_CORES,
                                    num_subcores=NUM_SUBCORES),
        scratch_types=[
            pltpu.VMEM((ROWS_PER * B,), jnp.float32),
            pltpu.VMEM((B + LANES,), jnp.int32),
            pltpu.VMEM((ROWS_PER * LANES,), jnp.float32),
            pltpu.VMEM((ROWS_PER * LANES,), jnp.float32),
        ],
    )


def _loss_body(mp_ref, mn_ref, out_ref):
    mp = mp_ref[...]
    mn = mn_ref[...]
    acc = jnp.zeros((128, 1), jnp.float32)
    for g in range(8):
        md2 = jnp.max(mp[:, g * 16:(g + 1) * 16], axis=1, keepdims=True)
        nd2 = jnp.min(mn[:, g * 16:(g + 1) * 16], axis=1, keepdims=True)
        dp = jnp.where(md2 > 1e-12,
                       jnp.sqrt(jnp.where(md2 > 1e-12, md2, 1.0)), 0.0)
        dn = jnp.where(nd2 > 1e-12,
                       jnp.sqrt(jnp.where(nd2 > 1e-12, nd2, 1.0)), 0.0)
        acc = acc + jnp.log1p(jnp.exp(dp - dn))
    out_ref[0, 0] = jnp.sum(acc)


def kernel(inputs, targets, W, b):
    d2 = pl.pallas_call(
        _mega_body,
        in_specs=[
            pl.BlockSpec(memory_space=pltpu.MemorySpace.HBM),
            pl.BlockSpec(memory_space=pltpu.MemorySpace.HBM),
            pl.BlockSpec(memory_space=pltpu.MemorySpace.HBM),
        ],
        out_specs=pl.BlockSpec(memory_space=pltpu.MemorySpace.HBM),
        out_shape=jax.ShapeDtypeStruct((B // 8, 8, 8, 128), jnp.float32),
        scratch_shapes=[
            pltpu.VMEM((B, D_IN), jnp.float32),
            pltpu.VMEM((D_IN, D_OUT), jnp.float32),
            pltpu.VMEM((1, D_OUT), jnp.float32),
            pltpu.VMEM((B, D_OUT), jnp.float32),
            pltpu.VMEM((BLK // 8, 8, 8, 128), jnp.float32),
            pltpu.VMEM((BLK // 8, 8, 8, 128), jnp.float32),
            pltpu.SemaphoreType.DMA((NRB,)),
            pltpu.SemaphoreType.DMA,
            pltpu.SemaphoreType.DMA,
            pltpu.SemaphoreType.DMA((2,)),
        ],
    )(inputs, W, b.reshape(1, D_OUT))

    mp, mn = _mine_kernel()(d2.reshape(B * B), targets)

    loss = pl.pallas_call(
        _loss_body,
        out_shape=jax.ShapeDtypeStruct((1, 1), jnp.float32),
        out_specs=pl.BlockSpec(memory_space=pltpu.SMEM),
    )(mp.reshape(128, 128), mn.reshape(128, 128))
    return loss.reshape(())


# speedup vs baseline: 1.2790x; 1.0317x over previous
import functools

import jax
import jax.numpy as jnp
from jax import lax
from jax.experimental import pallas as pl
from jax.experimental.pallas import tpu as pltpu
from jax.experimental.pallas import tpu_sc as plsc

B = 1024
D_IN = 2048
D_OUT = 256

NUM_CORES = 2
NUM_SUBCORES = 16
LANES = 16
NW = NUM_CORES * NUM_SUBCORES
ROWS_PER = B // NW
CHUNKS = B // LANES


BLK = 256
NRB = B // BLK


def _mega_body(x_hbm, w_hbm, b_hbm, out_hbm,
               x_v, w_v, b_v, h_v, buf0, buf1,
               sems_x, sem_w, sem_b, sems_o):
    cw = pltpu.make_async_copy(w_hbm, w_v, sem_w)
    cw.start()
    cb = pltpu.make_async_copy(b_hbm, b_v, sem_b)
    cb.start()
    cxs = []
    for c in range(NRB):
        cx = pltpu.make_async_copy(x_hbm.at[pl.ds(c * BLK, BLK)],
                                   x_v.at[pl.ds(c * BLK, BLK)],
                                   sems_x.at[c])
        cx.start()
        cxs.append(cx)
    cw.wait()
    cb.wait()
    bias = b_v[...]
    for c in range(NRB):
        cxs[c].wait()
        h_v[pl.ds(c * BLK, BLK), :] = (
            jnp.dot(x_v[pl.ds(c * BLK, BLK), :], w_v[...],
                    preferred_element_type=jnp.float32) + bias
        )
    h = h_v[...]
    hm = h * -2.0
    hh = h * h
    ones_row = jnp.ones((1, D_OUT), jnp.float32)
    sq_row = lax.dot_general(ones_row, hh, (((1,), (1,)), ((), ())),
                             preferred_element_type=jnp.float32)
    bufs = (buf0, buf1)
    outcps = []
    for rb in range(NRB):
        buf = bufs[rb % 2]
        if rb >= 2:
            outcps[rb - 2].wait()
        hb = hm[rb * BLK:(rb + 1) * BLK, :]
        sq_blk = jnp.sum(h[rb * BLK:(rb + 1) * BLK, :] ** 2, axis=1,
                         keepdims=True)
        for t in range(NRB):
            hc = h[t * BLK:(t + 1) * BLK, :]
            g = lax.dot_general(hb, hc, (((1,), (1,)), ((), ())),
                                preferred_element_type=jnp.float32)
            d2p = jnp.maximum(sq_blk + (sq_row[:, t * BLK:(t + 1) * BLK] + g),
                              0.0)
            for u in range(BLK // 128):
                buf[:, (t * BLK) // 128 + u, :, :] = (
                    d2p[:, u * 128:(u + 1) * 128].reshape(BLK // 8, 8, 128))
        cp = pltpu.make_async_copy(buf, out_hbm.at[pl.ds(rb * (BLK // 8),
                                                         BLK // 8)],
                                   sems_o.at[rb % 2])
        cp.start()
        outcps.append(cp)
    outcps[NRB - 2].wait()
    outcps[NRB - 1].wait()


RBLK = 4
NBLK = ROWS_PER // 8


def _mine_body(d2_hbm, tgt_hbm, mp_hbm, mn_hbm, d2_v, tgt_v, mp_v, mn_v,
               dsem):
    wid = lax.axis_index("s") * NUM_CORES + lax.axis_index("c")
    base = wid * ROWS_PER
    woff = base * B
    for bk in range(NBLK):
        pltpu.make_async_copy(d2_hbm.at[pl.ds(woff + bk * 8 * B, 8 * B)],
                              d2_v.at[pl.ds(bk * 8 * B, 8 * B)],
                              dsem).start()
    pltpu.sync_copy(tgt_hbm, tgt_v.at[pl.ds(0, B)])

    def blk_body(blk, _):
        pltpu.make_async_copy(d2_hbm.at[pl.ds(woff + blk * 8 * B, 8 * B)],
                              d2_v.at[pl.ds(blk * 8 * B, 8 * B)],
                              dsem).wait()
        dbase = blk * (8 * B)
        for half in range(2):
            r0 = blk * 8 + half * RBLK
            ts = [
                jnp.full((LANES,), tgt_v[pl.ds(base + r0 + i, LANES)][0],
                         jnp.int32)
                for i in range(RBLK)
            ]
            mp = [jnp.full((LANES,), -jnp.inf, jnp.float32)] * RBLK
            mn = [jnp.full((LANES,), jnp.inf, jnp.float32)] * RBLK
            for j in range(CHUNKS):
                tv = tgt_v[pl.ds(j * LANES, LANES)]
                joff = (j // 8) * B + (j % 8) * LANES
                for i in range(RBLK):
                    dv = d2_v[pl.ds(dbase + (half * RBLK + i) * 128 + joff,
                                    LANES)]
                    same = tv == ts[i]
                    mp[i] = jnp.maximum(mp[i], jnp.where(same, dv, -jnp.inf))
                    mn[i] = jnp.minimum(mn[i], jnp.where(same, jnp.inf, dv))
            for i in range(RBLK):
                mp_v[pl.ds((r0 + i) * LANES, LANES)] = mp[i]
                mn_v[pl.ds((r0 + i) * LANES, LANES)] = mn[i]
        return 0

    lax.fori_loop(0, NBLK, blk_body, 0)
    pltpu.sync_copy(mp_v, mp_hbm.at[pl.ds(base * LANES, ROWS_PER * LANES)])
    pltpu.sync_copy(mn_v, mn_hbm.at[pl.ds(base * LANES, ROWS_PER * LANES)])


@functools.lru_cache(maxsize=1)
def _mine_kernel():
    return pl.kernel(
        _mine_body,
        out_type=(
            jax.ShapeDtypeStruct((B * LANES,), jnp.float32),
            jax.ShapeDtypeStruct((B * LANES,), jnp.float32),
        ),
        mesh=plsc.VectorSubcoreMesh(core_axis_name="c", subcore_axis_name="s",
                                    num_cores=NUM_CORES,
                                    num_subcores=NUM_SUBCORES),
        scratch_types=[
            pltpu.VMEM((ROWS_PER * B,), jnp.float32),
            pltpu.VMEM((B + LANES,), jnp.int32),
            pltpu.VMEM((ROWS_PER * LANES,), jnp.float32),
            pltpu.VMEM((ROWS_PER * LANES,), jnp.float32),
            pltpu.SemaphoreType.DMA,
        ],
    )


def _loss_body(mp_ref, mn_ref, out_ref):
    mp = mp_ref[...]
    mn = mn_ref[...]
    acc = jnp.zeros((128, 1), jnp.float32)
    for g in range(8):
        md2 = jnp.max(mp[:, g * 16:(g + 1) * 16], axis=1, keepdims=True)
        nd2 = jnp.min(mn[:, g * 16:(g + 1) * 16], axis=1, keepdims=True)
        dp = jnp.where(md2 > 1e-12,
                       jnp.sqrt(jnp.where(md2 > 1e-12, md2, 1.0)), 0.0)
        dn = jnp.where(nd2 > 1e-12,
                       jnp.sqrt(jnp.where(nd2 > 1e-12, nd2, 1.0)), 0.0)
        acc = acc + jnp.log1p(jnp.exp(dp - dn))
    out_ref[0, 0] = jnp.sum(acc)


def kernel(inputs, targets, W, b):
    d2 = pl.pallas_call(
        _mega_body,
        in_specs=[
            pl.BlockSpec(memory_space=pltpu.MemorySpace.HBM),
            pl.BlockSpec(memory_space=pltpu.MemorySpace.HBM),
            pl.BlockSpec(memory_space=pltpu.MemorySpace.HBM),
        ],
        out_specs=pl.BlockSpec(memory_space=pltpu.MemorySpace.HBM),
        out_shape=jax.ShapeDtypeStruct((B // 8, 8, 8, 128), jnp.float32),
        scratch_shapes=[
            pltpu.VMEM((B, D_IN), jnp.float32),
            pltpu.VMEM((D_IN, D_OUT), jnp.float32),
            pltpu.VMEM((1, D_OUT), jnp.float32),
            pltpu.VMEM((B, D_OUT), jnp.float32),
            pltpu.VMEM((BLK // 8, 8, 8, 128), jnp.float32),
            pltpu.VMEM((BLK // 8, 8, 8, 128), jnp.float32),
            pltpu.SemaphoreType.DMA((NRB,)),
            pltpu.SemaphoreType.DMA,
            pltpu.SemaphoreType.DMA,
            pltpu.SemaphoreType.DMA((2,)),
        ],
    )(inputs, W, b.reshape(1, D_OUT))

    mp, mn = _mine_kernel()(d2.reshape(B * B), targets)

    loss = pl.pallas_call(
        _loss_body,
        out_shape=jax.ShapeDtypeStruct((1, 1), jnp.float32),
        out_specs=pl.BlockSpec(memory_space=pltpu.SMEM),
    )(mp.reshape(128, 128), mn.reshape(128, 128))
    return loss.reshape(())


# speedup vs baseline: 1.3207x; 1.0326x over previous
import functools

import jax
import jax.numpy as jnp
from jax import lax
from jax.experimental import pallas as pl
from jax.experimental.pallas import tpu as pltpu
from jax.experimental.pallas import tpu_sc as plsc

B = 1024
D_IN = 2048
D_OUT = 256

NUM_CORES = 2
NUM_SUBCORES = 16
LANES = 16
NW = NUM_CORES * NUM_SUBCORES
ROWS_PER = B // NW
CHUNKS = B // LANES


BLK = 256
NRB = B // BLK


def _mega_body(x_hbm, w_hbm, b_hbm, out_hbm,
               x_v, w_v, b_v, h_v, buf0, buf1,
               sems_x, sem_w, sem_b, sems_o):
    cw = pltpu.make_async_copy(w_hbm, w_v, sem_w)
    cw.start()
    cb = pltpu.make_async_copy(b_hbm, b_v, sem_b)
    cb.start()
    def x_copy(c):
        return pltpu.make_async_copy(x_hbm.at[pl.ds(c * BLK, BLK)],
                                     x_v.at[pl.ds(c * BLK, BLK)],
                                     sems_x.at[c])
    cxs = [x_copy(c) for c in range(NRB)]
    cxs[0].start()
    cxs[1].start()
    cw.wait()
    cb.wait()
    bias = b_v[...]
    for c in range(NRB):
        cxs[c].wait()
        if c + 2 < NRB:
            cxs[c + 2].start()
        h_v[pl.ds(c * BLK, BLK), :] = (
            jnp.dot(x_v[pl.ds(c * BLK, BLK), :], w_v[...],
                    preferred_element_type=jnp.float32) + bias
        )
    h = h_v[...]
    hm = h * -2.0
    hh = h * h
    ones_row = jnp.ones((1, D_OUT), jnp.float32)
    sq_row = lax.dot_general(ones_row, hh, (((1,), (1,)), ((), ())),
                             preferred_element_type=jnp.float32)
    bufs = (buf0, buf1)
    outcps = []
    for rb in range(NRB):
        buf = bufs[rb % 2]
        if rb >= 2:
            outcps[rb - 2].wait()
        hb = hm[rb * BLK:(rb + 1) * BLK, :]
        sq_blk = jnp.sum(h[rb * BLK:(rb + 1) * BLK, :] ** 2, axis=1,
                         keepdims=True)
        for t in range(NRB):
            hc = h[t * BLK:(t + 1) * BLK, :]
            g = lax.dot_general(hb, hc, (((1,), (1,)), ((), ())),
                                preferred_element_type=jnp.float32)
            d2p = jnp.maximum(sq_blk + (sq_row[:, t * BLK:(t + 1) * BLK] + g),
                              0.0)
            for u in range(BLK // 128):
                buf[:, (t * BLK) // 128 + u, :, :] = (
                    d2p[:, u * 128:(u + 1) * 128].reshape(BLK // 8, 8, 128))
        cp = pltpu.make_async_copy(buf, out_hbm.at[pl.ds(rb * (BLK // 8),
                                                         BLK // 8)],
                                   sems_o.at[rb % 2])
        cp.start()
        outcps.append(cp)
    outcps[NRB - 2].wait()
    outcps[NRB - 1].wait()


RBLK = 4
NBLK = ROWS_PER // 8


def _mine_body(d2_hbm, tgt_hbm, mp_hbm, mn_hbm, d2_v, tgt_v, mp_v, mn_v,
               dsem):
    wid = lax.axis_index("s") * NUM_CORES + lax.axis_index("c")
    base = wid * ROWS_PER
    woff = base * B
    for bk in range(NBLK):
        pltpu.make_async_copy(d2_hbm.at[pl.ds(woff + bk * 8 * B, 8 * B)],
                              d2_v.at[pl.ds(bk * 8 * B, 8 * B)],
                              dsem).start()
    pltpu.sync_copy(tgt_hbm, tgt_v.at[pl.ds(0, B)])

    def blk_body(blk, _):
        pltpu.make_async_copy(d2_hbm.at[pl.ds(woff + blk * 8 * B, 8 * B)],
                              d2_v.at[pl.ds(blk * 8 * B, 8 * B)],
                              dsem).wait()
        dbase = blk * (8 * B)
        for half in range(2):
            r0 = blk * 8 + half * RBLK
            ts = [
                jnp.full((LANES,), tgt_v[pl.ds(base + r0 + i, LANES)][0],
                         jnp.int32)
                for i in range(RBLK)
            ]
            mp = [jnp.full((LANES,), -jnp.inf, jnp.float32)] * RBLK
            mn = [jnp.full((LANES,), jnp.inf, jnp.float32)] * RBLK
            for j in range(CHUNKS):
                tv = tgt_v[pl.ds(j * LANES, LANES)]
                joff = (j // 8) * B + (j % 8) * LANES
                for i in range(RBLK):
                    dv = d2_v[pl.ds(dbase + (half * RBLK + i) * 128 + joff,
                                    LANES)]
                    same = tv == ts[i]
                    mp[i] = jnp.maximum(mp[i], jnp.where(same, dv, -jnp.inf))
                    mn[i] = jnp.minimum(mn[i], jnp.where(same, jnp.inf, dv))
            for i in range(RBLK):
                mp_v[pl.ds((r0 + i) * LANES, LANES)] = mp[i]
                mn_v[pl.ds((r0 + i) * LANES, LANES)] = mn[i]
        return 0

    lax.fori_loop(0, NBLK, blk_body, 0)
    pltpu.sync_copy(mp_v, mp_hbm.at[pl.ds(base * LANES, ROWS_PER * LANES)])
    pltpu.sync_copy(mn_v, mn_hbm.at[pl.ds(base * LANES, ROWS_PER * LANES)])


@functools.lru_cache(maxsize=1)
def _mine_kernel():
    return pl.kernel(
        _mine_body,
        out_type=(
            jax.ShapeDtypeStruct((B * LANES,), jnp.float32),
            jax.ShapeDtypeStruct((B * LANES,), jnp.float32),
        ),
        mesh=plsc.VectorSubcoreMesh(core_axis_name="c", subcore_axis_name="s",
                                    num_cores=NUM_CORES,
                                    num_subcores=NUM_SUBCORES),
        scratch_types=[
            pltpu.VMEM((ROWS_PER * B,), jnp.float32),
            pltpu.VMEM((B + LANES,), jnp.int32),
            pltpu.VMEM((ROWS_PER * LANES,), jnp.float32),
            pltpu.VMEM((ROWS_PER * LANES,), jnp.float32),
            pltpu.SemaphoreType.DMA,
        ],
    )


def _loss_body(mp_ref, mn_ref, out_ref):
    mp = mp_ref[...]
    mn = mn_ref[...]
    acc = jnp.zeros((128, 1), jnp.float32)
    for g in range(8):
        md2 = jnp.max(mp[:, g * 16:(g + 1) * 16], axis=1, keepdims=True)
        nd2 = jnp.min(mn[:, g * 16:(g + 1) * 16], axis=1, keepdims=True)
        dp = jnp.where(md2 > 1e-12,
                       jnp.sqrt(jnp.where(md2 > 1e-12, md2, 1.0)), 0.0)
        dn = jnp.where(nd2 > 1e-12,
                       jnp.sqrt(jnp.where(nd2 > 1e-12, nd2, 1.0)), 0.0)
        acc = acc + jnp.log1p(jnp.exp(dp - dn))
    out_ref[0, 0] = jnp.sum(acc)


def kernel(inputs, targets, W, b):
    d2 = pl.pallas_call(
        _mega_body,
        in_specs=[
            pl.BlockSpec(memory_space=pltpu.MemorySpace.HBM),
            pl.BlockSpec(memory_space=pltpu.MemorySpace.HBM),
            pl.BlockSpec(memory_space=pltpu.MemorySpace.HBM),
        ],
        out_specs=pl.BlockSpec(memory_space=pltpu.MemorySpace.HBM),
        out_shape=jax.ShapeDtypeStruct((B // 8, 8, 8, 128), jnp.float32),
        scratch_shapes=[
            pltpu.VMEM((B, D_IN), jnp.float32),
            pltpu.VMEM((D_IN, D_OUT), jnp.float32),
            pltpu.VMEM((1, D_OUT), jnp.float32),
            pltpu.VMEM((B, D_OUT), jnp.float32),
            pltpu.VMEM((BLK // 8, 8, 8, 128), jnp.float32),
            pltpu.VMEM((BLK // 8, 8, 8, 128), jnp.float32),
            pltpu.SemaphoreType.DMA((NRB,)),
            pltpu.SemaphoreType.DMA,
            pltpu.SemaphoreType.DMA,
            pltpu.SemaphoreType.DMA((2,)),
        ],
    )(inputs, W, b.reshape(1, D_OUT))

    mp, mn = _mine_kernel()(d2.reshape(B * B), targets)

    loss = pl.pallas_call(
        _loss_body,
        out_shape=jax.ShapeDtypeStruct((1, 1), jnp.float32),
        out_specs=pl.BlockSpec(memory_space=pltpu.SMEM),
    )(mp.reshape(128, 128), mn.reshape(128, 128))
    return loss.reshape(())
